# baseline (speedup 1.0000x reference)
import functools

import jax
import jax.numpy as jnp
from jax import lax
from jax.experimental import pallas as pl
from jax.experimental.pallas import tpu as pltpu
from jax.experimental.pallas import tpu_sc as plsc

_N, _C, _H, _W = 8, 19, 512, 512
_NP = _N * _H * _W
_K = 4096
_CH = 128
_NCH = _NP // _CH
_CPR = _NCH + 16
_NC, _NS, _L = 2, 16, 16
_NW = _NC * _NS
_PPW = _NP // _NW
_SB = 512
_NSTEP = _PPW // _SB
_JB = _K // _NW
_BH = 64

_mesh = plsc.VectorSubcoreMesh(core_axis_name="c", subcore_axis_name="s")
_sc_params = pltpu.CompilerParams(needs_layout_passes=False)


def _cumsum_minor(x):
    row = lax.broadcasted_iota(jnp.int32, (_CH, _CH), 0)
    col = lax.broadcasted_iota(jnp.int32, (_CH, _CH), 1)
    tri = (row <= col).astype(jnp.float32)
    return lax.dot_general(x, tri, (((x.ndim - 1,), (0,)), ((), ())),
                           precision=lax.Precision.HIGHEST,
                           preferred_element_type=jnp.float32)


def _softmax_body(x_ref, lp_ref, cs_ref):
    x = x_ref[0]
    m = jnp.max(x, axis=0, keepdims=True)
    e = jnp.exp(x - m)
    s = jnp.sum(e, axis=0, keepdims=True)
    p = e / s
    lp = _cumsum_minor(p.reshape(_C, _BH, _W // _CH, _CH))
    lp_ref[:, 0] = lp.reshape(_C, _BH, _W)
    cs_ref[:, 0] = lp[:, :, :, _CH - 1]


def _stage_a(logits):
    return pl.pallas_call(
        _softmax_body,
        grid=(_N, _H // _BH),
        in_specs=[pl.BlockSpec((1, _C, _BH, _W), lambda n, h: (n, 0, h, 0))],
        out_specs=[
            pl.BlockSpec((_C, 1, _BH, _W), lambda n, h: (0, n, h, 0)),
            pl.BlockSpec((_C, 1, _BH, _W // _CH), lambda n, h: (0, n, h, 0)),
        ],
        out_shape=[
            jax.ShapeDtypeStruct((_C, _N, _H, _W), jnp.float32),
            jax.ShapeDtypeStruct((_C, _N, _H, _W // _CH), jnp.float32),
        ],
    )(logits)


@functools.partial(
    pl.kernel,
    out_type=jax.ShapeDtypeStruct((_NW, _C * _K), jnp.int32),
    mesh=_mesh,
    compiler_params=_sc_params,
    scratch_types=[
        pltpu.VMEM((_C * _SB,), jnp.float32),
        pltpu.VMEM((_C * _SB,), jnp.float32),
        pltpu.VMEM((_C * _K,), jnp.int32),
        pltpu.SemaphoreType.DMA,
        pltpu.SemaphoreType.DMA,
    ],
)
def _hist_kernel(lp_hbm, out_hbm, buf0, buf1, hist, sem0, sem1):
    wid = lax.axis_index("s") * _NC + lax.axis_index("c")
    base = wid * _PPW

    @pl.loop(0, _C * _K // _L, unroll=8)
    def _zero(i):
        hist[pl.ds(i * _L, _L)] = jnp.zeros((_L,), jnp.int32)

    def _start(step, buf, sem):
        for c in range(_C):
            pltpu.async_copy(
                lp_hbm.at[pl.ds(pl.multiple_of(
                    c * _NP + base + step * _SB, _SB), _SB)],
                buf.at[pl.ds(c * _SB, _SB)], sem)

    def _wait(buf, sem):
        for c in range(_C):
            pltpu.make_async_copy(
                lp_hbm.at[pl.ds(0, _SB)], buf.at[pl.ds(c * _SB, _SB)], sem).wait()

    ones = jnp.ones((_L,), jnp.int32)
    iota = lax.iota(jnp.int32, _L)
    nz = iota > 0
    gidx = jnp.maximum(iota - 1, 0)
    i15 = jnp.full((_L,), 15, jnp.int32)

    def _process(buf):
        for c in range(_C):
            @pl.loop(0, _SB // (_L * 8))
            def _b(j, c=c, buf=buf):
                vals = [buf[pl.ds(c * _SB + j * (_L * 8) + i * _L, _L)]
                        for i in range(8)]
                idxs = []
                for i in range(8):
                    a = vals[i]
                    g = a.at[gidx].get(mode="promise_in_bounds")
                    if i == 0:
                        b = jnp.where(nz, g, jnp.float32(0))
                    else:
                        p15 = vals[i - 1].at[i15].get(
                            mode="promise_in_bounds")
                        b = jnp.where(nz, g, p15)
                    p = a - b
                    k = jnp.minimum((p * _K).astype(jnp.int32), _K - 1)
                    idxs.append(k + (c * _K))
                for k in idxs:
                    plsc.addupdate_scatter(hist, [k], ones)

    _start(0, buf0, sem0)
    _start(1, buf1, sem1)

    @pl.loop(0, _NSTEP // 2)
    def _steps(i2):
        s0 = i2 * 2
        _wait(buf0, sem0)
        _process(buf0)

        @pl.when(s0 + 2 < _NSTEP)
        def _():
            _start(s0 + 2, buf0, sem0)

        _wait(buf1, sem1)
        _process(buf1)

        @pl.when(s0 + 3 < _NSTEP)
        def _():
            _start(s0 + 3, buf1, sem1)

    pltpu.sync_copy(hist, out_hbm.at[wid])


@functools.partial(
    pl.kernel,
    out_type=(
        jax.ShapeDtypeStruct((_C, _K), jnp.int32),
        jax.ShapeDtypeStruct((_C, _CPR), jnp.float32),
    ),
    mesh=_mesh,
    compiler_params=_sc_params,
    scratch_types=[
        pltpu.VMEM((_NW, 1024), jnp.int32),
        pltpu.VMEM((_K,), jnp.int32),
        pltpu.VMEM((_K,), jnp.int32),
        pltpu.VMEM((_NCH,), jnp.float32),
        pltpu.VMEM((_CPR,), jnp.float32),
    ],
)
def _scan_kernel(hist_hbm, csums_hbm, bnd_hbm, cp_hbm, mbuf, cnt, bnd, csb, cpb):
    wid = lax.axis_index("s") * _NC + lax.axis_index("c")

    @pl.when(wid < _C)
    def _():
        c = wid
        for q in range(4):
            pltpu.sync_copy(hist_hbm.at[:, pl.ds(c * _K + q * 1024, 1024)],
                            mbuf)

            @pl.loop(0, 1024 // _L)
            def _m(j, q=q):
                acc = jnp.zeros((_L,), jnp.int32)
                for p in range(_NW):
                    acc = acc + mbuf[p, pl.ds(j * _L, _L)]
                cnt[pl.ds(q * 1024 + j * _L, _L)] = acc

        def _bstep(i, carry):
            v = lax.rev(cnt[pl.ds(_K - _L - i * _L, _L)], (0,))
            cs = plsc.cumsum(v)
            bnd[pl.ds(i * _L, _L)] = cs - v + carry
            return carry + jnp.sum(v)

        lax.fori_loop(0, _K // _L, _bstep, jnp.int32(0))
        pltpu.sync_copy(bnd, bnd_hbm.at[c])

        pltpu.sync_copy(csums_hbm.at[c], csb)

        def _cstep(i, carry):
            v = csb[pl.ds(i * _L, _L)]
            cs = plsc.cumsum(v)
            cpb[pl.ds(i * _L, _L)] = cs - v + carry
            return carry + jnp.sum(v)

        tot = lax.fori_loop(0, _NCH // _L, _cstep, jnp.float32(0))
        cpb[pl.ds(_NCH, _L)] = jnp.full((_L,), tot, jnp.float32)
        pltpu.sync_copy(cpb, cp_hbm.at[c])


@functools.partial(
    pl.kernel,
    out_type=jax.ShapeDtypeStruct((_NW, _L), jnp.float32),
    mesh=_mesh,
    compiler_params=_sc_params,
    scratch_types=[
        pltpu.VMEM((_JB,), jnp.int32),
        pltpu.VMEM((_JB,), jnp.int32),
        pltpu.VMEM((_JB,), jnp.int32),
        pltpu.VMEM((_JB,), jnp.int32),
        pltpu.VMEM((_JB,), jnp.float32),
        pltpu.VMEM((_JB,), jnp.float32),
        pltpu.VMEM((_L,), jnp.float32),
        pltpu.SemaphoreType.DMA,
    ],
)
def _gather_kernel(lp_hbm, bnd_hbm, cp_hbm, out_hbm,
                   bb, lidx, cpidx, rb, lpv, cpv, accb, sem):
    wid = lax.axis_index("s") * _NC + lax.axis_index("c")
    jbase = wid * _JB

    def _cls(c, acc):
        pltpu.sync_copy(
            bnd_hbm.at[c, pl.ds(pl.multiple_of(jbase, _JB), _JB)], bb)

        @pl.loop(0, _JB // _L)
        def _ix(j, c=c):
            b = bb[pl.ds(j * _L, _L)]
            chunk = jnp.right_shift(b, 7)
            r = jnp.bitwise_and(b, _CH - 1)
            rb[pl.ds(j * _L, _L)] = r
            cpidx[pl.ds(j * _L, _L)] = chunk + c * _CPR
            lidx[pl.ds(j * _L, _L)] = (
                jnp.maximum(jnp.minimum(b - 1, _NP - 1), 0) + c * _NP)

        pltpu.async_copy(cp_hbm.at[cpidx], cpv, sem).wait()
        pltpu.async_copy(lp_hbm.at[lidx], lpv, sem).wait()

        def _acc(j, a):
            r = rb[pl.ds(j * _L, _L)]
            local = jnp.where(r > 0, lpv[pl.ds(j * _L, _L)], jnp.float32(0))
            return a + cpv[pl.ds(j * _L, _L)] + local

        return lax.fori_loop(0, _JB // _L, _acc, acc)

    acc = lax.fori_loop(0, _C, _cls, jnp.zeros((_L,), jnp.float32))
    accb[...] = acc
    pltpu.sync_copy(accb, out_hbm.at[wid])


def kernel(logits, label):
    del label
    lp4, cs4 = _stage_a(logits)
    lp = lp4.reshape(-1)
    hist_parts = _hist_kernel(lp)
    bnd, cp = _scan_kernel(hist_parts, cs4.reshape(_C, _NCH))
    partials = _gather_kernel(lp, bnd, cp.reshape(-1))
    return (jnp.sum(partials) + 0.5 * _NP) / _K


# speedup vs baseline: 1.1331x; 1.1331x over previous
import functools

import jax
import jax.numpy as jnp
from jax import lax
from jax.experimental import pallas as pl
from jax.experimental.pallas import tpu as pltpu
from jax.experimental.pallas import tpu_sc as plsc

_N, _C, _H, _W = 8, 19, 512, 512
_NP = _N * _H * _W
_K = 4096
_CH = 128
_NCH = _NP // _CH
_CPR = _NCH + 16
_NC, _NS, _L = 2, 16, 16
_NW = _NC * _NS
_PPW = _NP // _NW
_SB = 1024
_NSTEP = _PPW // _SB
_JB = _K // _NW
_BH = 64

_mesh = plsc.VectorSubcoreMesh(core_axis_name="c", subcore_axis_name="s")
_sc_params = pltpu.CompilerParams(needs_layout_passes=False)


def _cumsum_minor(x):
    row = lax.broadcasted_iota(jnp.int32, (_CH, _CH), 0)
    col = lax.broadcasted_iota(jnp.int32, (_CH, _CH), 1)
    tri = (row <= col).astype(jnp.float32)
    return lax.dot_general(x, tri, (((x.ndim - 1,), (0,)), ((), ())),
                           precision=lax.Precision.HIGHEST,
                           preferred_element_type=jnp.float32)


def _softmax_body(x_ref, lp_ref, cs_ref):
    x = x_ref[0]
    m = jnp.max(x, axis=0, keepdims=True)
    e = jnp.exp(x - m)
    s = jnp.sum(e, axis=0, keepdims=True)
    p = e / s
    lp = _cumsum_minor(p.reshape(_C, _BH, _W // _CH, _CH))
    lp_ref[:, 0] = lp.reshape(_C, _BH, _W)
    cs_ref[:, 0] = lp[:, :, :, _CH - 1]


def _stage_a(logits):
    return pl.pallas_call(
        _softmax_body,
        grid=(_N, _H // _BH),
        in_specs=[pl.BlockSpec((1, _C, _BH, _W), lambda n, h: (n, 0, h, 0))],
        out_specs=[
            pl.BlockSpec((_C, 1, _BH, _W), lambda n, h: (0, n, h, 0)),
            pl.BlockSpec((_C, 1, _BH, _W // _CH), lambda n, h: (0, n, h, 0)),
        ],
        out_shape=[
            jax.ShapeDtypeStruct((_C, _N, _H, _W), jnp.float32),
            jax.ShapeDtypeStruct((_C, _N, _H, _W // _CH), jnp.float32),
        ],
    )(logits)


@functools.partial(
    pl.kernel,
    out_type=jax.ShapeDtypeStruct((_NW, _C * _K), jnp.int32),
    mesh=_mesh,
    compiler_params=_sc_params,
    scratch_types=[
        pltpu.VMEM((_C * _SB,), jnp.float32),
        pltpu.VMEM((_C * _SB,), jnp.float32),
        pltpu.VMEM((_C * _K,), jnp.int32),
        pltpu.SemaphoreType.DMA,
        pltpu.SemaphoreType.DMA,
    ],
)
def _hist_kernel(lp_hbm, out_hbm, buf0, buf1, hist, sem0, sem1):
    wid = lax.axis_index("s") * _NC + lax.axis_index("c")
    base = wid * _PPW

    @pl.loop(0, _C * _K // _L, unroll=8)
    def _zero(i):
        hist[pl.ds(i * _L, _L)] = jnp.zeros((_L,), jnp.int32)

    def _start(step, buf, sem):
        for c in range(_C):
            pltpu.async_copy(
                lp_hbm.at[pl.ds(pl.multiple_of(
                    c * _NP + base + step * _SB, _SB), _SB)],
                buf.at[pl.ds(c * _SB, _SB)], sem)

    def _wait(buf, sem):
        for c in range(_C):
            pltpu.make_async_copy(
                lp_hbm.at[pl.ds(0, _SB)], buf.at[pl.ds(c * _SB, _SB)], sem).wait()

    ones = jnp.ones((_L,), jnp.int32)
    iota = lax.iota(jnp.int32, _L)
    nz = iota > 0
    gidx = jnp.maximum(iota - 1, 0)
    i15 = jnp.full((_L,), 15, jnp.int32)

    def _process(buf):
        for c in range(_C):
            @pl.loop(0, _SB // (_L * 8))
            def _b(j, c=c, buf=buf):
                vals = [buf[pl.ds(c * _SB + j * (_L * 8) + i * _L, _L)]
                        for i in range(8)]
                idxs = []
                for i in range(8):
                    a = vals[i]
                    g = a.at[gidx].get(mode="promise_in_bounds")
                    if i == 0:
                        b = jnp.where(nz, g, jnp.float32(0))
                    else:
                        p15 = vals[i - 1].at[i15].get(
                            mode="promise_in_bounds")
                        b = jnp.where(nz, g, p15)
                    p = a - b
                    k = jnp.minimum((p * _K).astype(jnp.int32), _K - 1)
                    idxs.append(k + (c * _K))
                for k in idxs:
                    plsc.addupdate_scatter(hist, [k], ones)

    _start(0, buf0, sem0)
    _start(1, buf1, sem1)

    @pl.loop(0, _NSTEP // 2)
    def _steps(i2):
        s0 = i2 * 2
        _wait(buf0, sem0)
        _process(buf0)

        @pl.when(s0 + 2 < _NSTEP)
        def _():
            _start(s0 + 2, buf0, sem0)

        _wait(buf1, sem1)
        _process(buf1)

        @pl.when(s0 + 3 < _NSTEP)
        def _():
            _start(s0 + 3, buf1, sem1)

    pltpu.sync_copy(hist, out_hbm.at[wid])


@functools.partial(
    pl.kernel,
    out_type=(
        jax.ShapeDtypeStruct((_C, _K), jnp.int32),
        jax.ShapeDtypeStruct((_C, _CPR), jnp.float32),
    ),
    mesh=_mesh,
    compiler_params=_sc_params,
    scratch_types=[
        pltpu.VMEM((_NW, 1024), jnp.int32),
        pltpu.VMEM((_K,), jnp.int32),
        pltpu.VMEM((_K,), jnp.int32),
        pltpu.VMEM((_NCH,), jnp.float32),
        pltpu.VMEM((_CPR,), jnp.float32),
    ],
)
def _scan_kernel(hist_hbm, csums_hbm, bnd_hbm, cp_hbm, mbuf, cnt, bnd, csb, cpb):
    wid = lax.axis_index("s") * _NC + lax.axis_index("c")

    @pl.when(wid < _C)
    def _():
        c = wid
        for q in range(4):
            pltpu.sync_copy(hist_hbm.at[:, pl.ds(c * _K + q * 1024, 1024)],
                            mbuf)

            @pl.loop(0, 1024 // _L)
            def _m(j, q=q):
                acc = jnp.zeros((_L,), jnp.int32)
                for p in range(_NW):
                    acc = acc + mbuf[p, pl.ds(j * _L, _L)]
                cnt[pl.ds(q * 1024 + j * _L, _L)] = acc

        def _bstep(i, carry):
            v = lax.rev(cnt[pl.ds(_K - _L - i * _L, _L)], (0,))
            cs = plsc.cumsum(v)
            bnd[pl.ds(i * _L, _L)] = cs - v + carry
            return carry + jnp.sum(v)

        lax.fori_loop(0, _K // _L, _bstep, jnp.int32(0))
        pltpu.sync_copy(bnd, bnd_hbm.at[c])

        pltpu.sync_copy(csums_hbm.at[c], csb)

        def _cstep(i, carry):
            v = csb[pl.ds(i * _L, _L)]
            cs = plsc.cumsum(v)
            cpb[pl.ds(i * _L, _L)] = cs - v + carry
            return carry + jnp.sum(v)

        tot = lax.fori_loop(0, _NCH // _L, _cstep, jnp.float32(0))
        cpb[pl.ds(_NCH, _L)] = jnp.full((_L,), tot, jnp.float32)
        pltpu.sync_copy(cpb, cp_hbm.at[c])


@functools.partial(
    pl.kernel,
    out_type=jax.ShapeDtypeStruct((_NW, _L), jnp.float32),
    mesh=_mesh,
    compiler_params=_sc_params,
    scratch_types=[
        pltpu.VMEM((_JB,), jnp.int32),
        pltpu.VMEM((_JB,), jnp.int32),
        pltpu.VMEM((_JB,), jnp.int32),
        pltpu.VMEM((_JB,), jnp.int32),
        pltpu.VMEM((_JB,), jnp.float32),
        pltpu.VMEM((_JB,), jnp.float32),
        pltpu.VMEM((_L,), jnp.float32),
        pltpu.SemaphoreType.DMA,
    ],
)
def _gather_kernel(lp_hbm, bnd_hbm, cp_hbm, out_hbm,
                   bb, lidx, cpidx, rb, lpv, cpv, accb, sem):
    wid = lax.axis_index("s") * _NC + lax.axis_index("c")
    jbase = wid * _JB

    def _cls(c, acc):
        pltpu.sync_copy(
            bnd_hbm.at[c, pl.ds(pl.multiple_of(jbase, _JB), _JB)], bb)

        @pl.loop(0, _JB // _L)
        def _ix(j, c=c):
            b = bb[pl.ds(j * _L, _L)]
            chunk = jnp.right_shift(b, 7)
            r = jnp.bitwise_and(b, _CH - 1)
            rb[pl.ds(j * _L, _L)] = r
            cpidx[pl.ds(j * _L, _L)] = chunk + c * _CPR
            lidx[pl.ds(j * _L, _L)] = (
                jnp.maximum(jnp.minimum(b - 1, _NP - 1), 0) + c * _NP)

        pltpu.async_copy(cp_hbm.at[cpidx], cpv, sem).wait()
        pltpu.async_copy(lp_hbm.at[lidx], lpv, sem).wait()

        def _acc(j, a):
            r = rb[pl.ds(j * _L, _L)]
            local = jnp.where(r > 0, lpv[pl.ds(j * _L, _L)], jnp.float32(0))
            return a + cpv[pl.ds(j * _L, _L)] + local

        return lax.fori_loop(0, _JB // _L, _acc, acc)

    acc = lax.fori_loop(0, _C, _cls, jnp.zeros((_L,), jnp.float32))
    accb[...] = acc
    pltpu.sync_copy(accb, out_hbm.at[wid])


def kernel(logits, label):
    del label
    lp4, cs4 = _stage_a(logits)
    lp = lp4.reshape(-1)
    hist_parts = _hist_kernel(lp)
    bnd, cp = _scan_kernel(hist_parts, cs4.reshape(_C, _NCH))
    partials = _gather_kernel(lp, bnd, cp.reshape(-1))
    return (jnp.sum(partials) + 0.5 * _NP) / _K
